# baseline (speedup 1.0000x reference)
import functools

import jax
import jax.numpy as jnp
from jax import lax
from jax.experimental import pallas as pl
from jax.experimental.pallas import tpu as pltpu
from jax.experimental.pallas import tpu_sc as plsc

_NUM_CORES = 2
_NUM_SUBCORES = 16
_NW = _NUM_CORES * _NUM_SUBCORES

_C = 200


def _sc_gather_mul_packed(x32, ep_flat, p):
    n, dw = x32.shape
    per_w = p // _NW
    n_chunks = per_w // _C
    assert p % _NW == 0 and per_w % _C == 0 and n_chunks % 2 == 0
    n2 = n_chunks // 2
    c2w = _C // 2

    mesh = plsc.VectorSubcoreMesh(core_axis_name="c", subcore_axis_name="s")

    @functools.partial(
        pl.kernel,
        out_type=jax.ShapeDtypeStruct((p // 2, 2 * dw), jnp.int32),
        mesh=mesh,
        compiler_params=pltpu.CompilerParams(use_tc_tiling_on_sc=False),
        scratch_types=[
            pltpu.VMEM((p // _NW,), jnp.int32),
            pltpu.VMEM((p // _NW,), jnp.int32),
            pltpu.VMEM((2, _C, dw), jnp.int32),
            pltpu.VMEM((2, _C, dw), jnp.int32),
            pltpu.VMEM((2, _C // 2, 2 * dw), jnp.int32),
            pltpu.SemaphoreType.DMA,
            pltpu.SemaphoreType.DMA,
            pltpu.SemaphoreType.DMA,
            pltpu.SemaphoreType.DMA,
        ],
    )
    def k(x_hbm, ep_hbm, out_hbm, ii_all, jj_all, xi_v, xj_v, o_v,
          g0, g1, s0, s1):
        wid = lax.axis_index("s") * _NUM_CORES + lax.axis_index("c")
        base = wid * per_w
        gsem = (g0, g1)
        ssem = (s0, s1)
        pltpu.sync_copy(ep_hbm.at[pl.ds(base, per_w)], ii_all)
        pltpu.sync_copy(ep_hbm.at[pl.ds(p + base, per_w)], jj_all)

        def fire(off, b):
            loc = off - base
            pltpu.async_copy(
                x_hbm.at[ii_all.at[pl.ds(loc, _C)]], xi_v.at[b], gsem[b])
            pltpu.async_copy(
                x_hbm.at[jj_all.at[pl.ds(loc, _C)]], xj_v.at[b], gsem[b])

        def drain_gathers(b):
            pltpu.make_async_copy(
                x_hbm.at[ii_all.at[pl.ds(0, _C)]], xi_v.at[b], gsem[b]).wait()
            pltpu.make_async_copy(
                x_hbm.at[jj_all.at[pl.ds(0, _C)]], xj_v.at[b], gsem[b]).wait()

        def drain_scatter(b):
            pltpu.make_async_copy(
                o_v.at[b], out_hbm.at[pl.ds(0, c2w)], ssem[b]).wait()

        def mult(b):
            mask = jnp.int32(-65536)
            half = jnp.int32(0x8000)

            def rowpair(r2, carry):
                for rr in range(2):
                    r = r2 * 2 + rr
                    for v in range(dw // 16):
                        sl = pl.ds(v * 16, 16)
                        vi = xi_v[b, r, sl]
                        vj = xj_v[b, r, sl]
                        ae = lax.bitcast_convert_type(vi << 16, jnp.float32)
                        ao = lax.bitcast_convert_type(vi & mask, jnp.float32)
                        be = lax.bitcast_convert_type(vj << 16, jnp.float32)
                        bo = lax.bitcast_convert_type(vj & mask, jnp.float32)
                        pe = lax.bitcast_convert_type(ae * be, jnp.int32)
                        po = lax.bitcast_convert_type(ao * bo, jnp.int32)
                        lo = lax.shift_right_logical(pe + half, 16)
                        hi = (po + half) & mask
                        o_v[b, r2, pl.ds(rr * dw + v * 16, 16)] = lo | hi
                return carry
            lax.fori_loop(0, c2w, rowpair, 0, unroll=2)

        def scatter(off, b):
            pltpu.async_copy(
                o_v.at[b], out_hbm.at[pl.ds(off // 2, c2w)], ssem[b])

        fire(base, 0)

        def body2(c2, carry):
            c = c2 * 2
            off0 = base + c * _C
            fire(off0 + _C, 1)
            drain_gathers(0)

            @pl.when(c2 > 0)
            def _():
                drain_scatter(0)
            mult(0)
            scatter(off0, 0)
            @pl.when(c2 + 1 < n2)
            def _():
                fire(off0 + 2 * _C, 0)
            drain_gathers(1)

            @pl.when(c2 > 0)
            def _():
                drain_scatter(1)
            mult(1)
            scatter(off0 + _C, 1)
            return carry

        lax.fori_loop(0, n2, body2, 0)
        drain_scatter(0)
        drain_scatter(1)

    return k(x32, ep_flat)


def _tc_mlp_packed(h2, W1, b1, W2, b2):
    p2, d = h2.shape
    dw = d // 2
    blk = 1280
    grid = p2 // blk
    assert p2 % blk == 0
    w1e = W1[0::2, :]
    w1o = W1[1::2, :]
    zed = jnp.zeros_like(w1e)

    def blockdiag(a, b):
        return jnp.concatenate(
            [jnp.concatenate([a, jnp.zeros_like(a)], axis=1),
             jnp.concatenate([jnp.zeros_like(b), b], axis=1)], axis=0)

    w1e_bd = blockdiag(w1e, w1e).astype(jnp.bfloat16)
    w1o_bd = blockdiag(w1o, w1o).astype(jnp.bfloat16)
    w2_bd = blockdiag(W2, W2)
    b1_2d = jnp.concatenate([b1, b1]).reshape(1, 2 * d)
    b2_2d = b2.reshape(1, 1)

    def body(h_ref, w1e_ref, w1o_ref, w2_ref, b1_ref, b2_ref, o_ref):
        hv = h_ref[...]
        e = lax.bitcast_convert_type(hv << 16, jnp.float32).astype(jnp.bfloat16)
        o = lax.bitcast_convert_type(
            hv & jnp.int32(-65536), jnp.float32).astype(jnp.bfloat16)
        z = (jnp.dot(e, w1e_ref[...], preferred_element_type=jnp.float32)
             + jnp.dot(o, w1o_ref[...], preferred_element_type=jnp.float32))
        z = jnp.maximum(z + b1_ref[...], 0.0)
        t = jnp.dot(z, w2_ref[...], preferred_element_type=jnp.float32)
        t = t + b2_ref[...]
        o_ref[...] = 1.0 / (1.0 + jnp.exp(-t))

    return pl.pallas_call(
        body,
        grid=(grid,),
        in_specs=[
            pl.BlockSpec((blk, d), lambda i: (i, 0)),
            pl.BlockSpec((d, 2 * d), lambda i: (0, 0)),
            pl.BlockSpec((d, 2 * d), lambda i: (0, 0)),
            pl.BlockSpec((2 * d, 2), lambda i: (0, 0)),
            pl.BlockSpec((1, 2 * d), lambda i: (0, 0)),
            pl.BlockSpec((1, 1), lambda i: (0, 0)),
        ],
        out_specs=pl.BlockSpec((blk, 2), lambda i: (i, 0)),
        out_shape=jax.ShapeDtypeStruct((p2, 2), jnp.float32),
    )(h2, w1e_bd, w1o_bd, w2_bd, b1_2d, b2_2d)


def kernel(x, edge_index, edge_pairs, W1, b1, W2, b2):
    del edge_index
    n, d = x.shape
    p = edge_pairs.shape[1]
    x32 = lax.bitcast_convert_type(
        x.astype(jnp.bfloat16).reshape(n, d // 2, 2), jnp.int32)
    ep_flat = edge_pairs.reshape(-1)
    h2 = _sc_gather_mul_packed(x32, ep_flat, p)
    out2 = _tc_mlp_packed(h2, W1, b1, W2, b2)
    return out2.reshape(p, 1)


# speedup vs baseline: 1.5856x; 1.5856x over previous
import functools

import jax
import jax.numpy as jnp
from jax import lax
from jax.experimental import pallas as pl
from jax.experimental.pallas import tpu as pltpu
from jax.experimental.pallas import tpu_sc as plsc

_NUM_CORES = 2
_NUM_SUBCORES = 16
_NW = _NUM_CORES * _NUM_SUBCORES

_C = 200


def _sc_gather_mul_packed(x32, ep_flat, p):
    n, dw = x32.shape
    per_w = p // _NW
    n_chunks = per_w // _C
    assert p % _NW == 0 and per_w % _C == 0 and n_chunks % 2 == 0
    n2 = n_chunks // 2
    c2w = _C // 2

    mesh = plsc.VectorSubcoreMesh(core_axis_name="c", subcore_axis_name="s")

    @functools.partial(
        pl.kernel,
        out_type=jax.ShapeDtypeStruct((p // 2, 2 * dw), jnp.int32),
        mesh=mesh,
        compiler_params=pltpu.CompilerParams(use_tc_tiling_on_sc=False),
        scratch_types=[
            pltpu.VMEM((p // _NW,), jnp.int32),
            pltpu.VMEM((p // _NW,), jnp.int32),
            pltpu.VMEM((2, _C, dw), jnp.int32),
            pltpu.VMEM((2, _C, dw), jnp.int32),
            pltpu.VMEM((2, _C // 2, 2 * dw), jnp.int32),
            pltpu.SemaphoreType.DMA,
            pltpu.SemaphoreType.DMA,
            pltpu.SemaphoreType.DMA,
            pltpu.SemaphoreType.DMA,
        ],
    )
    def k(x_hbm, ep_hbm, out_hbm, ii_all, jj_all, xi_v, xj_v, o_v,
          g0, g1, s0, s1):
        wid = lax.axis_index("s") * _NUM_CORES + lax.axis_index("c")
        base = wid * per_w
        gsem = (g0, g1)
        ssem = (s0, s1)
        pltpu.sync_copy(ep_hbm.at[pl.ds(base, per_w)], ii_all)
        pltpu.sync_copy(ep_hbm.at[pl.ds(p + base, per_w)], jj_all)

        def fire(off, b):
            loc = off - base
            pltpu.async_copy(
                x_hbm.at[ii_all.at[pl.ds(loc, _C)]], xi_v.at[b], gsem[b])
            pltpu.async_copy(
                x_hbm.at[jj_all.at[pl.ds(loc, _C)]], xj_v.at[b], gsem[b])

        def drain_gathers(b):
            pltpu.make_async_copy(
                x_hbm.at[ii_all.at[pl.ds(0, _C)]], xi_v.at[b], gsem[b]).wait()
            pltpu.make_async_copy(
                x_hbm.at[jj_all.at[pl.ds(0, _C)]], xj_v.at[b], gsem[b]).wait()

        def drain_scatter(b):
            pltpu.make_async_copy(
                o_v.at[b], out_hbm.at[pl.ds(0, c2w)], ssem[b]).wait()

        def mult(b):
            mask = jnp.int32(-65536)
            half = jnp.int32(0x8000)

            def rowpair(r2, carry):
                for rr in range(2):
                    r = r2 * 2 + rr
                    for v in range(dw // 16):
                        sl = pl.ds(v * 16, 16)
                        vi = xi_v[b, r, sl]
                        vj = xj_v[b, r, sl]
                        ae = lax.bitcast_convert_type(vi << 16, jnp.float32)
                        ao = lax.bitcast_convert_type(vi & mask, jnp.float32)
                        be = lax.bitcast_convert_type(vj << 16, jnp.float32)
                        bo = lax.bitcast_convert_type(vj & mask, jnp.float32)
                        pe = lax.bitcast_convert_type(ae * be, jnp.int32)
                        po = lax.bitcast_convert_type(ao * bo, jnp.int32)
                        lo = lax.shift_right_logical(pe + half, 16)
                        hi = (po + half) & mask
                        o_v[b, r2, pl.ds(rr * dw + v * 16, 16)] = lo | hi
                return carry
            lax.fori_loop(0, c2w, rowpair, 0)

        def scatter(off, b):
            pltpu.async_copy(
                o_v.at[b], out_hbm.at[pl.ds(off // 2, c2w)], ssem[b])

        fire(base, 0)

        def body2(c2, carry):
            c = c2 * 2
            off0 = base + c * _C
            fire(off0 + _C, 1)
            drain_gathers(0)

            @pl.when(c2 > 0)
            def _():
                drain_scatter(0)
            mult(0)
            scatter(off0, 0)
            @pl.when(c2 + 1 < n2)
            def _():
                fire(off0 + 2 * _C, 0)
            drain_gathers(1)

            @pl.when(c2 > 0)
            def _():
                drain_scatter(1)
            mult(1)
            scatter(off0 + _C, 1)
            return carry

        lax.fori_loop(0, n2, body2, 0)
        drain_scatter(0)
        drain_scatter(1)

    return k(x32, ep_flat)


def _tc_mlp_packed(h2, W1, b1, W2, b2):
    p2, d = h2.shape
    dw = d // 2
    blk = 2000
    grid = p2 // blk
    assert p2 % blk == 0
    w1e = W1[0::2, :]
    w1o = W1[1::2, :]
    zed = jnp.zeros_like(w1e)

    def blockdiag(a, b):
        return jnp.concatenate(
            [jnp.concatenate([a, jnp.zeros_like(a)], axis=1),
             jnp.concatenate([jnp.zeros_like(b), b], axis=1)], axis=0)

    w1e_bd = blockdiag(w1e, w1e)
    w1o_bd = blockdiag(w1o, w1o)
    w2_bd = blockdiag(W2, W2)
    b1_2d = jnp.concatenate([b1, b1]).reshape(1, 2 * d)
    b2_2d = b2.reshape(1, 1)

    def body(h_ref, w1e_ref, w1o_ref, w2_ref, b1_ref, b2_ref, o_ref):
        hv = h_ref[...]
        e = lax.bitcast_convert_type(hv << 16, jnp.float32)
        o = lax.bitcast_convert_type(hv & jnp.int32(-65536), jnp.float32)
        z = (jnp.dot(e, w1e_ref[...], preferred_element_type=jnp.float32)
             + jnp.dot(o, w1o_ref[...], preferred_element_type=jnp.float32))
        z = jnp.maximum(z + b1_ref[...], 0.0)
        t = jnp.dot(z, w2_ref[...], preferred_element_type=jnp.float32)
        t = t + b2_ref[...]
        o_ref[...] = 1.0 / (1.0 + jnp.exp(-t))

    return pl.pallas_call(
        body,
        grid=(grid,),
        in_specs=[
            pl.BlockSpec((blk, d), lambda i: (i, 0)),
            pl.BlockSpec((d, 2 * d), lambda i: (0, 0)),
            pl.BlockSpec((d, 2 * d), lambda i: (0, 0)),
            pl.BlockSpec((2 * d, 2), lambda i: (0, 0)),
            pl.BlockSpec((1, 2 * d), lambda i: (0, 0)),
            pl.BlockSpec((1, 1), lambda i: (0, 0)),
        ],
        out_specs=pl.BlockSpec((blk, 2), lambda i: (i, 0)),
        out_shape=jax.ShapeDtypeStruct((p2, 2), jnp.float32),
    )(h2, w1e_bd, w1o_bd, w2_bd, b1_2d, b2_2d)


def kernel(x, edge_index, edge_pairs, W1, b1, W2, b2):
    del edge_index
    n, d = x.shape
    p = edge_pairs.shape[1]
    x32 = lax.bitcast_convert_type(
        x.astype(jnp.bfloat16).reshape(n, d // 2, 2), jnp.int32)
    ep_flat = edge_pairs.reshape(-1)
    h2 = _sc_gather_mul_packed(x32, ep_flat, p)
    out2 = _tc_mlp_packed(h2, W1, b1, W2, b2)
    return out2.reshape(p, 1)
